# baseline (speedup 1.0000x reference)
import functools

import jax
import jax.numpy as jnp
from jax import lax
from jax.experimental import pallas as pl
from jax.experimental.pallas import tpu as pltpu
from jax.experimental.pallas import tpu_sc as plsc

_NC = 2
_NS = 16
_NW = _NC * _NS
_CH = 125


def _sc_gather(table, idx3):
    nw, nch, ch = idx3.shape
    ring, ahead = 7, 4
    full_blocks = nch // ring
    rem = nch % ring
    assert full_blocks >= 3 and 0 < rem < ahead
    rows_total = nw * nch * ch
    nb, n, d = table.shape
    per_w = nch * ch
    w_per_b = nw // nb
    assert w_per_b * nb == nw and per_w * w_per_b == n
    mesh = plsc.VectorSubcoreMesh(core_axis_name="c", subcore_axis_name="s")

    @functools.partial(
        pl.kernel,
        mesh=mesh,
        out_type=jax.ShapeDtypeStruct((rows_total, d), jnp.float32),
        scratch_types=[
            pltpu.VMEM((nch, ch), jnp.int32),
            [pltpu.VMEM((ch, d), jnp.float32) for _ in range(ring)],
            [pltpu.SemaphoreType.DMA for _ in range(ring)],
            [pltpu.SemaphoreType.DMA for _ in range(ring)],
        ],
        compiler_params=pltpu.CompilerParams(use_tc_tiling_on_sc=False),
    )
    def gather_kernel(table_hbm, idx_hbm, out_hbm, idx_v, rows, gsem, ssem):
        wid = lax.axis_index("s") * _NC + lax.axis_index("c")
        pltpu.sync_copy(idx_hbm.at[wid], idx_v)
        base = wid * per_w
        batch = wid // w_per_b

        def fire_gather(j, b):
            pltpu.async_copy(
                table_hbm.at[batch].at[idx_v.at[j]], rows[b], gsem[b])

        def wait_gather(b):
            pltpu.make_async_copy(
                table_hbm.at[0].at[pl.ds(0, ch)], rows[b], gsem[b]).wait()

        def fire_store(j, b):
            pltpu.async_copy(
                rows[b], out_hbm.at[pl.ds(base + j * ch, ch)], ssem[b])

        def wait_store(b):
            pltpu.make_async_copy(
                rows[b], out_hbm.at[pl.ds(0, ch)], ssem[b]).wait()

        for f in range(ahead):
            fire_gather(f, f)

        for b in range(ring):
            wait_gather(b)
            fire_store(b, b)
            t = (b + ahead) % ring
            if b + ahead >= ring:
                wait_store(t)
            fire_gather(b + ahead, t)

        def step(k, carry):
            for b in range(ring):
                j = ring * k + b
                wait_gather(b)
                fire_store(j, b)
                t = (b + ahead) % ring
                wait_store(t)
                fire_gather(j + ahead, t)
            return carry

        lax.fori_loop(1, full_blocks - 1, step, 0)

        for b in range(ring):
            j = ring * (full_blocks - 1) + b
            wait_gather(b)
            fire_store(j, b)
            if j + ahead < nch:
                t = (b + ahead) % ring
                wait_store(t)
                fire_gather(j + ahead, t)

        for r in range(rem):
            j = ring * full_blocks + r
            wait_gather(j % ring)
            fire_store(j, j % ring)
        for j in range(nch - ring, nch):
            wait_store(j % ring)

    return gather_kernel(table, idx3)


def kernel(vertices, reverse_map):
    b, n, d = vertices.shape
    nch = (b * n) // (_NW * _CH)
    idx3 = reverse_map.astype(jnp.int32).reshape(_NW, nch, _CH)
    out = _sc_gather(vertices, idx3)
    return out.reshape(b, n, d)
